# Initial kernel scaffold; baseline (speedup 1.0000x reference)
import jax
import jax.numpy as jnp
from jax.experimental import pallas as pl


def kernel(x, edge_index, edge_type_in, edge_attr, W_in, b_in, msg_W0, rel_emb0, rel_proj_W0, att_vec0, bias0, ln_g0, ln_b0, msg_W1, rel_emb1, rel_proj_W1, att_vec1, bias1, ln_g1, ln_b1):
    raise NotImplementedError("write your pallas kernel here")


# speedup vs baseline: 14.7740x; 14.7740x over previous
import functools

import jax
import jax.numpy as jnp
from jax import lax
from jax.experimental import pallas as pl
from jax.experimental.pallas import tpu as pltpu
from jax.experimental.pallas import tpu_sc as plsc

N = 10000
E = 320000
IN_DIM = 128
HID = 64
NUM_RELS = 8
CLW = 0.5

NTILE = 32
EPT = E // NTILE
K = 80
NCHUNK = EPT // K
RPT = N // 16

_mesh = plsc.VectorSubcoreMesh(core_axis_name="c", subcore_axis_name="s")


def _sc_sweep_body(src_hbm, dst_hbm, tt_hbm, clw_hbm, sd_hbm, ss_hbm, sr_hbm,
                   hw_hbm, z64_hbm, z1_hbm,
                   outu_hbm, den_hbm,
                   sd_l, ss_l, sr_l, den_l, sidx, didx, tt_l, clw_l, p_l,
                   rows, acc, sem):
    c = lax.axis_index("c")
    s = lax.axis_index("s")
    wid = c * 16 + s
    pltpu.sync_copy(sd_hbm, sd_l)
    pltpu.sync_copy(ss_hbm, ss_l)
    pltpu.sync_copy(sr_hbm, sr_l)
    pltpu.sync_copy(z1_hbm, den_l)
    pltpu.sync_copy(z64_hbm.at[pl.ds(s * RPT, RPT)], acc.at[pl.ds(s * RPT, RPT)])
    plsc.subcore_barrier()

    base_e = wid * EPT

    @pl.loop(0, NCHUNK)
    def _chunk(i):
        off = base_e + i * K
        pltpu.sync_copy(src_hbm.at[pl.ds(off, K)], sidx)
        pltpu.sync_copy(dst_hbm.at[pl.ds(off, K)], didx)
        pltpu.sync_copy(tt_hbm.at[pl.ds(off, K)], tt_l)
        pltpu.sync_copy(clw_hbm.at[pl.ds(off, K)], clw_l)
        gather = pltpu.async_copy(hw_hbm.at[sidx], rows, sem)
        for g in range(K // 16):
            dv = didx[pl.ds(g * 16, 16)]
            sv = sidx[pl.ds(g * 16, 16)]
            tv = tt_l[pl.ds(g * 16, 16)]
            e = (plsc.load_gather(sd_l, [dv])
                 + plsc.load_gather(ss_l, [sv])
                 + plsc.load_gather(sr_l, [tv]))
            e = jnp.where(e >= 0.0, e, 0.2 * e) + clw_l[pl.ds(g * 16, 16)]
            p = jnp.exp(e)
            p_l[pl.ds(g * 16, 16)] = p
            plsc.addupdate_scatter(den_l, [dv], p)
        gather.wait()

        @pl.loop(0, K // 16)
        def _scale(g):
            pv16 = p_l[pl.ds(g * 16, 16)]
            for j2 in range(16):
                pv = jnp.full((16,), pv16[j2], jnp.float32)
                j = g * 16 + j2
                for cg in range(4):
                    rows[j, pl.ds(cg * 16, 16)] = (
                        rows[j, pl.ds(cg * 16, 16)] * pv)

        pltpu.sync_copy(rows, acc.at[didx], add=True)

    plsc.subcore_barrier()
    pltpu.sync_copy(acc.at[pl.ds(s * RPT, RPT)],
                    outu_hbm.at[c, pl.ds(s * RPT, RPT)])
    pltpu.sync_copy(den_l, den_hbm.at[wid])


_sc_sweep = functools.partial(
    pl.kernel,
    out_type=[
        jax.ShapeDtypeStruct((2, N, HID), jnp.float32),
        jax.ShapeDtypeStruct((NTILE, N), jnp.float32),
    ],
    mesh=_mesh,
    compiler_params=pltpu.CompilerParams(use_tc_tiling_on_sc=False,
                                         needs_layout_passes=False),
    scratch_types=[
        pltpu.VMEM((N,), jnp.float32),
        pltpu.VMEM((N,), jnp.float32),
        pltpu.VMEM((16,), jnp.float32),
        pltpu.VMEM((N,), jnp.float32),
        pltpu.VMEM((K,), jnp.int32),
        pltpu.VMEM((K,), jnp.int32),
        pltpu.VMEM((K,), jnp.int32),
        pltpu.VMEM((K,), jnp.float32),
        pltpu.VMEM((K,), jnp.float32),
        pltpu.VMEM((K, HID), jnp.float32),
        pltpu.VMEM_SHARED((N, HID), jnp.float32),
        pltpu.SemaphoreType.DMA,
    ],
)(_sc_sweep_body)


def _layernorm(x, g, b):
    mu = jnp.mean(x, axis=-1, keepdims=True)
    var = jnp.mean((x - mu) ** 2, axis=-1, keepdims=True)
    return (x - mu) / jnp.sqrt(var + 1e-5) * g + b


def _prep_tables(h, msg_W, rel_emb, rel_proj_W, att_vec):
    hw = jnp.dot(h, msg_W.T, preferred_element_type=jnp.float32)
    sd = jnp.dot(hw, att_vec[:HID], preferred_element_type=jnp.float32)
    ss = jnp.dot(hw, att_vec[HID:2 * HID], preferred_element_type=jnp.float32)
    rp = jnp.dot(rel_emb, rel_proj_W.T, preferred_element_type=jnp.float32)
    sr8 = jnp.dot(rp, att_vec[2 * HID:], preferred_element_type=jnp.float32)
    sr = jnp.concatenate([sr8, jnp.zeros((16 - NUM_RELS,), jnp.float32)])
    return hw, sd, ss, sr


def _prep0_body(x_ref, win_ref, bin_ref, mw_ref, re_ref, rp_ref, av_ref,
                et_ref, conf_ref,
                h_ref, hw_ref, sd_ref, ss_ref, sr_ref, tt_ref, clw_ref):
    h = jax.nn.relu(jnp.dot(x_ref[...], win_ref[...].T,
                            preferred_element_type=jnp.float32) + bin_ref[...])
    h_ref[...] = h
    hw, sd, ss, sr = _prep_tables(h, mw_ref[...], re_ref[...], rp_ref[...],
                                  av_ref[...])
    hw_ref[...] = hw
    sd_ref[...] = sd
    ss_ref[...] = ss
    sr_ref[...] = sr
    tt_ref[...] = jnp.clip(et_ref[...], 0, NUM_RELS - 1)
    clw_ref[...] = CLW * jnp.log(jnp.maximum(conf_ref[...], 1e-6))


_prep0 = pl.pallas_call(
    _prep0_body,
    out_shape=[
        jax.ShapeDtypeStruct((N, HID), jnp.float32),
        jax.ShapeDtypeStruct((N, HID), jnp.float32),
        jax.ShapeDtypeStruct((N,), jnp.float32),
        jax.ShapeDtypeStruct((N,), jnp.float32),
        jax.ShapeDtypeStruct((16,), jnp.float32),
        jax.ShapeDtypeStruct((E,), jnp.int32),
        jax.ShapeDtypeStruct((E,), jnp.float32),
    ],
)


def _finish_mid_body(h_ref, outu_ref, den_ref, bias_ref, g_ref, b_ref,
                     mw_ref, re_ref, rp_ref, av_ref,
                     hn_ref, hw_ref, sd_ref, ss_ref, sr_ref):
    den = jnp.sum(den_ref[...], axis=0)
    out = (outu_ref[0] + outu_ref[1]) / (den[:, None] + 1e-16) + bias_ref[...]
    hn = _layernorm(h_ref[...] + jax.nn.relu(out), g_ref[...], b_ref[...])
    hn_ref[...] = hn
    hw, sd, ss, sr = _prep_tables(hn, mw_ref[...], re_ref[...], rp_ref[...],
                                  av_ref[...])
    hw_ref[...] = hw
    sd_ref[...] = sd
    ss_ref[...] = ss
    sr_ref[...] = sr


_finish_mid = pl.pallas_call(
    _finish_mid_body,
    out_shape=[
        jax.ShapeDtypeStruct((N, HID), jnp.float32),
        jax.ShapeDtypeStruct((N, HID), jnp.float32),
        jax.ShapeDtypeStruct((N,), jnp.float32),
        jax.ShapeDtypeStruct((N,), jnp.float32),
        jax.ShapeDtypeStruct((16,), jnp.float32),
    ],
)


def _finish_last_body(h_ref, outu_ref, den_ref, bias_ref, g_ref, b_ref,
                      hn_ref):
    den = jnp.sum(den_ref[...], axis=0)
    out = (outu_ref[0] + outu_ref[1]) / (den[:, None] + 1e-16) + bias_ref[...]
    hn_ref[...] = _layernorm(h_ref[...] + jax.nn.relu(out), g_ref[...],
                             b_ref[...])


_finish_last = pl.pallas_call(
    _finish_last_body,
    out_shape=jax.ShapeDtypeStruct((N, HID), jnp.float32),
)


def kernel(x, edge_index, edge_type_in, edge_attr, W_in, b_in,
           msg_W0, rel_emb0, rel_proj_W0, att_vec0, bias0, ln_g0, ln_b0,
           msg_W1, rel_emb1, rel_proj_W1, att_vec1, bias1, ln_g1, ln_b1):
    src = edge_index[0]
    dst = edge_index[1]
    conf = edge_attr[:, 0]
    z64 = jnp.zeros((N, HID), jnp.float32)
    z1 = jnp.zeros((N,), jnp.float32)

    h, hw, sd, ss, sr, tt, clw = _prep0(
        x, W_in, b_in, msg_W0, rel_emb0, rel_proj_W0, att_vec0,
        edge_type_in, conf)

    outu, den = _sc_sweep(src, dst, tt, clw, sd, ss, sr, hw, z64, z1)
    h, hw, sd, ss, sr = _finish_mid(
        h, outu, den, bias0, ln_g0, ln_b0,
        msg_W1, rel_emb1, rel_proj_W1, att_vec1)

    outu, den = _sc_sweep(src, dst, tt, clw, sd, ss, sr, hw, z64, z1)
    return _finish_last(h, outu, den, bias1, ln_g1, ln_b1)


# speedup vs baseline: 32.5681x; 2.2044x over previous
import functools

import jax
import jax.numpy as jnp
from jax import lax
from jax.experimental import pallas as pl
from jax.experimental.pallas import tpu as pltpu
from jax.experimental.pallas import tpu_sc as plsc

N = 10000
E = 320000
IN_DIM = 128
HID = 64
NUM_RELS = 8
CLW = 0.5

NTILE = 32
EPT = E // NTILE
K = 80
NCHUNK = EPT // K
RPT = N // 16

_mesh = plsc.VectorSubcoreMesh(core_axis_name="c", subcore_axis_name="s")


NPAIR = (NCHUNK - 1) // 2


def _sc_sweep_body(src_hbm, dst_hbm, tt_hbm, clw_hbm, sd_hbm, ss_hbm, sr_hbm,
                   hw_hbm, z64_hbm, z1_hbm,
                   outu_hbm, den_hbm,
                   sd_l, ss_l, sr_l, den_l, src_a, dst_a, tt_a, clw_a, p_l,
                   rows0, rows1, acc,
                   gsem0, gsem1, ssem0, ssem1):
    c = lax.axis_index("c")
    s = lax.axis_index("s")
    wid = c * 16 + s
    pltpu.sync_copy(sd_hbm, sd_l)
    pltpu.sync_copy(ss_hbm, ss_l)
    pltpu.sync_copy(sr_hbm, sr_l)
    pltpu.sync_copy(z1_hbm, den_l)
    pltpu.sync_copy(src_hbm.at[wid], src_a)
    pltpu.sync_copy(dst_hbm.at[wid], dst_a)
    pltpu.sync_copy(tt_hbm.at[wid], tt_a)
    pltpu.sync_copy(clw_hbm.at[wid], clw_a)
    pltpu.sync_copy(z64_hbm.at[pl.ds(s * RPT, RPT)], acc.at[pl.ds(s * RPT, RPT)])
    plsc.subcore_barrier()

    def scalar_pass(i):
        for g in range(K // 16):
            off = i * K + g * 16
            dv = dst_a[i, pl.ds(g * 16, 16)]
            e = (plsc.load_gather(sd_l, [dv])
                 + plsc.load_gather(ss_l, [src_a[pl.ds(off, 16)]])
                 + plsc.load_gather(sr_l, [tt_a[pl.ds(off, 16)]]))
            e = jnp.where(e >= 0.0, e, 0.2 * e) + clw_a[pl.ds(off, 16)]
            p = jnp.exp(e)
            p_l[pl.ds(g * 16, 16)] = p
            plsc.addupdate_scatter(den_l, [dv], p)

    def scale(rows):
        for g in range(K // 16):
            pv16 = p_l[pl.ds(g * 16, 16)]
            for j2 in range(16):
                pv = jnp.full((16,), pv16[j2], jnp.float32)
                j = g * 16 + j2
                for cg in range(4):
                    rows[j, pl.ds(cg * 16, 16)] = (
                        rows[j, pl.ds(cg * 16, 16)] * pv)

    def gather(i, rows, gsem):
        return pltpu.make_async_copy(
            hw_hbm.at[src_a.at[pl.ds(i * K, K)]], rows, gsem)

    def scatter(i, rows, ssem):
        return pltpu.make_async_copy(rows, acc.at[dst_a.at[i]], ssem)

    pltpu.async_copy(hw_hbm.at[src_a.at[pl.ds(0, K)]], rows0, gsem0)

    @pl.loop(0, NPAIR)
    def _pair(q):
        i0 = 2 * q
        scalar_pass(i0)
        gather(i0, rows0, gsem0).wait()
        scale(rows0)
        @pl.when(q > 0)
        def _():
            scatter(i0 - 1, rows1, ssem1).wait()
        pltpu.async_copy(hw_hbm.at[src_a.at[pl.ds((i0 + 1) * K, K)]],
                         rows1, gsem1)
        pltpu.async_copy(rows0, acc.at[dst_a.at[i0]], ssem0, add=True)
        scalar_pass(i0 + 1)
        gather(i0 + 1, rows1, gsem1).wait()
        scale(rows1)
        scatter(i0, rows0, ssem0).wait()
        pltpu.async_copy(hw_hbm.at[src_a.at[pl.ds((i0 + 2) * K, K)]],
                         rows0, gsem0)
        pltpu.async_copy(rows1, acc.at[dst_a.at[i0 + 1]], ssem1, add=True)

    scalar_pass(NCHUNK - 1)
    gather(NCHUNK - 1, rows0, gsem0).wait()
    scale(rows0)
    scatter(NCHUNK - 2, rows1, ssem1).wait()
    pltpu.sync_copy(rows0, acc.at[dst_a.at[NCHUNK - 1]], add=True)

    plsc.subcore_barrier()
    pltpu.sync_copy(acc.at[pl.ds(s * RPT, RPT)],
                    outu_hbm.at[c, pl.ds(s * RPT, RPT)])
    pltpu.sync_copy(den_l, den_hbm.at[wid])


_sc_sweep = functools.partial(
    pl.kernel,
    out_type=[
        jax.ShapeDtypeStruct((2, N, HID), jnp.float32),
        jax.ShapeDtypeStruct((NTILE, N), jnp.float32),
    ],
    mesh=_mesh,
    compiler_params=pltpu.CompilerParams(use_tc_tiling_on_sc=False,
                                         needs_layout_passes=False),
    scratch_types=[
        pltpu.VMEM((N,), jnp.float32),
        pltpu.VMEM((N,), jnp.float32),
        pltpu.VMEM((16,), jnp.float32),
        pltpu.VMEM((N,), jnp.float32),
        pltpu.VMEM((EPT,), jnp.int32),
        pltpu.VMEM((NCHUNK, K), jnp.int32),
        pltpu.VMEM((EPT,), jnp.int32),
        pltpu.VMEM((EPT,), jnp.float32),
        pltpu.VMEM((K,), jnp.float32),
        pltpu.VMEM((K, HID), jnp.float32),
        pltpu.VMEM((K, HID), jnp.float32),
        pltpu.VMEM_SHARED((N, HID), jnp.float32),
        pltpu.SemaphoreType.DMA,
        pltpu.SemaphoreType.DMA,
        pltpu.SemaphoreType.DMA,
        pltpu.SemaphoreType.DMA,
    ],
)(_sc_sweep_body)


def _layernorm(x, g, b):
    mu = jnp.mean(x, axis=-1, keepdims=True)
    var = jnp.mean((x - mu) ** 2, axis=-1, keepdims=True)
    return (x - mu) / jnp.sqrt(var + 1e-5) * g + b


def _prep_tables(h, msg_W, rel_emb, rel_proj_W, att_vec):
    hw = jnp.dot(h, msg_W.T, preferred_element_type=jnp.float32)
    sd = jnp.dot(hw, att_vec[:HID], preferred_element_type=jnp.float32)
    ss = jnp.dot(hw, att_vec[HID:2 * HID], preferred_element_type=jnp.float32)
    rp = jnp.dot(rel_emb, rel_proj_W.T, preferred_element_type=jnp.float32)
    sr8 = jnp.dot(rp, att_vec[2 * HID:], preferred_element_type=jnp.float32)
    sr = jnp.concatenate([sr8, jnp.zeros((16 - NUM_RELS,), jnp.float32)])
    return hw, sd, ss, sr


def _prep0_body(x_ref, win_ref, bin_ref, mw_ref, re_ref, rp_ref, av_ref,
                et_ref, conf_ref,
                h_ref, hw_ref, sd_ref, ss_ref, sr_ref, tt_ref, clw_ref):
    h = jax.nn.relu(jnp.dot(x_ref[...], win_ref[...].T,
                            preferred_element_type=jnp.float32) + bin_ref[...])
    h_ref[...] = h
    hw, sd, ss, sr = _prep_tables(h, mw_ref[...], re_ref[...], rp_ref[...],
                                  av_ref[...])
    hw_ref[...] = hw
    sd_ref[...] = sd
    ss_ref[...] = ss
    sr_ref[...] = sr
    tt_ref[...] = jnp.clip(et_ref[...], 0, NUM_RELS - 1)
    clw_ref[...] = CLW * jnp.log(jnp.maximum(conf_ref[...], 1e-6))


_prep0 = pl.pallas_call(
    _prep0_body,
    out_shape=[
        jax.ShapeDtypeStruct((N, HID), jnp.float32),
        jax.ShapeDtypeStruct((N, HID), jnp.float32),
        jax.ShapeDtypeStruct((N,), jnp.float32),
        jax.ShapeDtypeStruct((N,), jnp.float32),
        jax.ShapeDtypeStruct((16,), jnp.float32),
        jax.ShapeDtypeStruct((E,), jnp.int32),
        jax.ShapeDtypeStruct((E,), jnp.float32),
    ],
)


def _finish_mid_body(h_ref, outu_ref, den_ref, bias_ref, g_ref, b_ref,
                     mw_ref, re_ref, rp_ref, av_ref,
                     hn_ref, hw_ref, sd_ref, ss_ref, sr_ref):
    den = jnp.sum(den_ref[...], axis=0)
    out = (outu_ref[0] + outu_ref[1]) / (den[:, None] + 1e-16) + bias_ref[...]
    hn = _layernorm(h_ref[...] + jax.nn.relu(out), g_ref[...], b_ref[...])
    hn_ref[...] = hn
    hw, sd, ss, sr = _prep_tables(hn, mw_ref[...], re_ref[...], rp_ref[...],
                                  av_ref[...])
    hw_ref[...] = hw
    sd_ref[...] = sd
    ss_ref[...] = ss
    sr_ref[...] = sr


_finish_mid = pl.pallas_call(
    _finish_mid_body,
    out_shape=[
        jax.ShapeDtypeStruct((N, HID), jnp.float32),
        jax.ShapeDtypeStruct((N, HID), jnp.float32),
        jax.ShapeDtypeStruct((N,), jnp.float32),
        jax.ShapeDtypeStruct((N,), jnp.float32),
        jax.ShapeDtypeStruct((16,), jnp.float32),
    ],
)


def _finish_last_body(h_ref, outu_ref, den_ref, bias_ref, g_ref, b_ref,
                      hn_ref):
    den = jnp.sum(den_ref[...], axis=0)
    out = (outu_ref[0] + outu_ref[1]) / (den[:, None] + 1e-16) + bias_ref[...]
    hn_ref[...] = _layernorm(h_ref[...] + jax.nn.relu(out), g_ref[...],
                             b_ref[...])


_finish_last = pl.pallas_call(
    _finish_last_body,
    out_shape=jax.ShapeDtypeStruct((N, HID), jnp.float32),
)


def kernel(x, edge_index, edge_type_in, edge_attr, W_in, b_in,
           msg_W0, rel_emb0, rel_proj_W0, att_vec0, bias0, ln_g0, ln_b0,
           msg_W1, rel_emb1, rel_proj_W1, att_vec1, bias1, ln_g1, ln_b1):
    src = edge_index[0].reshape(NTILE, EPT)
    dst = edge_index[1].reshape(NTILE, NCHUNK, K)
    conf = edge_attr[:, 0]
    z64 = jnp.zeros((N, HID), jnp.float32)
    z1 = jnp.zeros((N,), jnp.float32)

    h, hw, sd, ss, sr, tt, clw = _prep0(
        x, W_in, b_in, msg_W0, rel_emb0, rel_proj_W0, att_vec0,
        edge_type_in, conf)
    tt = tt.reshape(NTILE, EPT)
    clw = clw.reshape(NTILE, EPT)

    outu, den = _sc_sweep(src, dst, tt, clw, sd, ss, sr, hw, z64, z1)
    h, hw, sd, ss, sr = _finish_mid(
        h, outu, den, bias0, ln_g0, ln_b0,
        msg_W1, rel_emb1, rel_proj_W1, att_vec1)

    outu, den = _sc_sweep(src, dst, tt, clw, sd, ss, sr, hw, z64, z1)
    return _finish_last(h, outu, den, bias1, ln_g1, ln_b1)
